# baseline (speedup 1.0000x reference)
import dataclasses
import functools

import jax
import jax.numpy as jnp
from jax import lax
from jax.experimental import pallas as pl
from jax.experimental.pallas import tpu as pltpu
from jax.experimental.pallas import tpu_sc as plsc

_N = 10000
_E = 320000
_D = 128
_H = 5000
_C = 128
_NSUB = 16
_NBUF = 3
_G = 32
_NSUB_G = _C // _G
_GRP_Q = 3 * _NSUB * _C
_EPAD = _E + 4 * _GRP_Q
_NROWS = _EPAD // _C
_RPT = 312
_RPT_LAST = _H - 15 * _RPT


def _sc_scatter(swdat, ddat, xh, cnts, zslab):
    mesh = plsc.VectorSubcoreMesh(core_axis_name="c", subcore_axis_name="s")
    cp = pltpu.CompilerParams()
    if "needs_layout_passes" in pltpu.CompilerParams.__dataclass_fields__:
        cp = dataclasses.replace(cp, needs_layout_passes=False)

    @functools.partial(
        pl.kernel,
        out_type=jax.ShapeDtypeStruct((_N, _D), jnp.float32),
        mesh=mesh,
        compiler_params=cp,
        scratch_types=[
            pltpu.VMEM_SHARED((_H, _D), jnp.float32),
            pltpu.VMEM_SHARED((_H, _D), jnp.float32),
            pltpu.VMEM((2 * _NBUF, _C), jnp.int32),
            pltpu.VMEM((1, _C), jnp.int32),
            pltpu.VMEM((_NBUF, _C, _D), jnp.float32),
        ] + [pltpu.SemaphoreType.DMA] * ((2 + _NSUB_G) * _NBUF + 1),
    )
    def k(swdat_hbm, ddat_hbm, xh_hbm, cnt_hbm, z_hbm, out_hbm,
          acc, xsp, ebuf, dring, rows, *sems):
        ng = _NSUB_G * _NBUF
        gsem = sems[:ng]
        ssem = sems[ng:ng + _NBUF]
        esem = sems[ng + _NBUF:ng + 2 * _NBUF]
        dsem = sems[ng + 2 * _NBUF]
        c = lax.axis_index("c")
        s = lax.axis_index("s")
        slab = s * _RPT
        myrows = jnp.where(s == _NSUB - 1, _RPT_LAST, _RPT)

        pltpu.sync_copy(cnt_hbm.at[0], dring.at[0])

        csc = [plsc.load_gather(dring, [jnp.full((16,), 0, jnp.int32),
                                        jnp.full((16,), j, jnp.int32)])[0]
               for j in range(8)]

        @pl.when(s < _NSUB - 1)
        def _():
            pltpu.sync_copy(z_hbm.at[pl.ds(0, _RPT)],
                            acc.at[pl.ds(slab, _RPT)])

        @pl.when(s == _NSUB - 1)
        def _():
            pltpu.sync_copy(z_hbm, acc.at[pl.ds(15 * _RPT, _RPT_LAST)])

        def load_xsp(half):
            @pl.when(s < _NSUB - 1)
            def _():
                pltpu.sync_copy(xh_hbm.at[pl.ds(half * _H + slab, _RPT)],
                                xsp.at[pl.ds(slab, _RPT)])

            @pl.when(s == _NSUB - 1)
            def _():
                pltpu.sync_copy(xh_hbm.at[pl.ds(half * _H + 15 * _RPT,
                                                _RPT_LAST)],
                                xsp.at[pl.ds(15 * _RPT, _RPT_LAST)])

        def estart(row, b):
            pltpu.async_copy(swdat_hbm.at[row], ebuf.at[pl.ds(2 * b, 2)],
                             esem[b])

        def ewait(b):
            pltpu.make_async_copy(swdat_hbm.at[0], ebuf.at[pl.ds(0, 2)],
                                  esem[b]).wait()

        def gstart(b):
            for h in range(_NSUB_G):
                sl = pl.ds(h * _G, _G)
                pltpu.async_copy(xsp.at[ebuf.at[2 * b, sl]],
                                 rows.at[b, sl], gsem[_NSUB_G * b + h])

        def gwait(b):
            for h in range(_NSUB_G):
                sl = pl.ds(h * _G, _G)
                pltpu.make_async_copy(xh_hbm.at[pl.ds(0, _G)], rows.at[b, sl],
                                      gsem[_NSUB_G * b + h]).wait()

        def swait(b):
            pltpu.make_async_copy(xh_hbm.at[pl.ds(0, _C)],
                                  acc.at[pl.ds(0, _C)], ssem[b]).wait()

        def scale(b):
            @pl.loop(0, _C)
            def _(e):
                wv = plsc.bitcast(
                    plsc.load_gather(
                        ebuf, [jnp.full((16,), 2 * b + 1, jnp.int32),
                               jnp.full((16,), e, jnp.int32)]),
                    jnp.float32)
                for kk in range(8):
                    sl = (b, e, pl.ds(kk * 16, 16))
                    rows[sl] = rows[sl] * wv

        def sel4(vals, idx):
            r = vals[3]
            for j in (2, 1, 0):
                r = jnp.where(idx == j, vals[j], r)
            return r

        def run_pass(p):
            idx = 2 * p + c
            n = sel4(csc[0:4], idx)
            cbase = sel4(csc[4:8], idx)
            row0 = cbase + s * n

            @pl.when(n > 0)
            def _():
                for b in range(_NBUF):
                    estart(row0 + b, b)

                @pl.loop(0, n, step=_NBUF)
                def _(ci):
                    for b in range(_NBUF):
                        ewait(b)
                        gstart(b)
                    for b in range(_NBUF):
                        gwait(b)
                    pltpu.async_copy(ddat_hbm.at[row0 + ci], dring.at[0],
                                     dsem)
                    for b in range(_NBUF):
                        scale(b)

                        @pl.when(ci + b + _NBUF < n)
                        def _():
                            estart(row0 + ci + b + _NBUF, b)
                    for b in range(_NBUF):
                        pltpu.make_async_copy(ddat_hbm.at[0], dring.at[0],
                                              dsem).wait()
                        pltpu.sync_copy(rows.at[b], acc.at[dring.at[0]],
                                        add=True)
                        if b + 1 < _NBUF:
                            pltpu.async_copy(ddat_hbm.at[row0 + ci + b + 1],
                                             dring.at[0], dsem)

        @pl.loop(0, 2)
        def _(p):
            load_xsp(jnp.where(p == 0, c, 1 - c))
            plsc.subcore_barrier()
            run_pass(p)
            plsc.subcore_barrier()

        out0 = c * _H

        @pl.when(s < _NSUB - 1)
        def _():
            pltpu.sync_copy(acc.at[pl.ds(slab, _RPT)],
                            out_hbm.at[pl.ds(out0 + slab, _RPT)])

        @pl.when(s == _NSUB - 1)
        def _():
            pltpu.sync_copy(acc.at[pl.ds(15 * _RPT, _RPT_LAST)],
                            out_hbm.at[pl.ds(out0 + 15 * _RPT, _RPT_LAST)])

    return k(swdat, ddat, xh, cnts, zslab)


_BLK = 1000


def _tc_finish(x, grads, Wt, b2):
    def body(x_ref, g_ref, wt_ref, b_ref, o_ref):
        g = g_ref[...]
        nrm = jnp.sqrt(jnp.sum(g * g, axis=1, keepdims=True))
        gn = g / jnp.maximum(nrm, 1e-12)
        h = jnp.concatenate([x_ref[...], gn], axis=1)
        o = lax.dot_general(h, wt_ref[...], (((1,), (0,)), ((), ())),
                            preferred_element_type=jnp.float32,
                            precision=lax.Precision.HIGHEST) + b_ref[...]
        nrm2 = jnp.sqrt(jnp.sum(o * o, axis=1, keepdims=True))
        o_ref[...] = o / jnp.maximum(nrm2, 1e-12)

    return pl.pallas_call(
        body,
        grid=(_N // _BLK,),
        in_specs=[
            pl.BlockSpec((_BLK, _D), lambda i: (i, 0)),
            pl.BlockSpec((_BLK, _D), lambda i: (i, 0)),
            pl.BlockSpec((2 * _D, _D), lambda i: (0, 0)),
            pl.BlockSpec((1, _D), lambda i: (0, 0)),
        ],
        out_specs=pl.BlockSpec((_BLK, _D), lambda i: (i, 0)),
        out_shape=jax.ShapeDtypeStruct((_N, _D), jnp.float32),
    )(x, grads, Wt, b2)


def kernel(x, edge_index, edge_weight, W, b):
    src = edge_index[0]
    dst = edge_index[1]
    sh = (src >= _H).astype(jnp.int32)
    dh = (dst >= _H).astype(jnp.int32)
    grp = sh + 2 * dh
    src_l = src - _H * sh
    dst_l = dst - _H * dh
    wbits = lax.bitcast_convert_type(edge_weight, jnp.int32)

    ranks = []
    sizes = []
    for g in range(4):
        m = (grp == g).astype(jnp.int32)
        cs = jnp.cumsum(m)
        ranks.append(cs - 1)
        sizes.append(cs[-1])
    sizes = jnp.stack(sizes)
    padded = ((sizes + (_GRP_Q - 1)) // _GRP_Q) * _GRP_Q
    bases = jnp.concatenate([jnp.zeros((1,), jnp.int32),
                             jnp.cumsum(padded)[:3].astype(jnp.int32)])
    rank = jnp.select([grp == g for g in range(4)], ranks)
    dest = bases[grp] + rank

    ssrc = jnp.zeros((_EPAD,), jnp.int32).at[dest].set(src_l)
    sdst = jnp.zeros((_EPAD,), jnp.int32).at[dest].set(dst_l)
    sw = jnp.zeros((_EPAD,), jnp.int32).at[dest].set(wbits)
    swdat = jnp.stack([ssrc.reshape(_NROWS, _C), sw.reshape(_NROWS, _C)],
                      axis=1)
    ddat = sdst.reshape(_NROWS, _C)

    gmap = jnp.array([0, 3, 1, 2], jnp.int32)
    ntile = (padded[gmap] // (_NSUB * _C)).astype(jnp.int32)
    cbase = (bases[gmap] // _C).astype(jnp.int32)
    crow = jnp.zeros((_C,), jnp.int32)
    crow = crow.at[0:4].set(ntile)
    crow = crow.at[4:8].set(cbase)
    cnts = jnp.tile(crow[None, :], (8, 1))

    xh = x.reshape(2 * _H, _D)
    zslab = jnp.zeros((_RPT_LAST, _D), jnp.float32)
    grads = _sc_scatter(swdat, ddat, xh, cnts, zslab)
    return _tc_finish(x, grads, W.T, b[None, :])


# speedup vs baseline: 7.1761x; 7.1761x over previous
import dataclasses
import functools

import jax
import jax.numpy as jnp
from jax import lax
from jax.experimental import pallas as pl
from jax.experimental.pallas import tpu as pltpu
from jax.experimental.pallas import tpu_sc as plsc

_N = 10000
_E = 320000
_D = 128
_C = 128
_NSUB = 16
_NTILES = 32
_CPT = 79
_EPAD = _C * _NTILES * _CPT
_NPAD = 10112
_RPT = _NPAD // _NSUB


def _sc_scatter(src2, dst2, w2, x, zslab):
    mesh = plsc.VectorSubcoreMesh(core_axis_name="c", subcore_axis_name="s")
    cp = pltpu.CompilerParams()
    if "needs_layout_passes" in pltpu.CompilerParams.__dataclass_fields__:
        cp = dataclasses.replace(cp, needs_layout_passes=False)

    @functools.partial(
        pl.kernel,
        out_type=jax.ShapeDtypeStruct((2 * _NPAD, _D), jnp.float32),
        mesh=mesh,
        compiler_params=cp,
        scratch_types=[
            pltpu.VMEM_SHARED((_NPAD, _D), jnp.float32),
            pltpu.VMEM((1, _C), jnp.int32),
            pltpu.VMEM((1, _C), jnp.int32),
            pltpu.VMEM((_C,), jnp.float32),
            pltpu.VMEM((_C, _D), jnp.float32),
            pltpu.SemaphoreType.DMA,
        ],
    )
    def k(src_hbm, dst_hbm, w_hbm, x_hbm, z_hbm, out_hbm,
          acc, sidx, didx, wbuf, rows, sem):
        c = lax.axis_index("c")
        s = lax.axis_index("s")
        wid = c * _NSUB + s
        slab = s * _RPT

        pltpu.sync_copy(z_hbm, acc.at[pl.ds(slab, _RPT)])
        plsc.subcore_barrier()

        base_chunk = wid * _CPT

        @pl.loop(0, _CPT)
        def _(ci):
            row = base_chunk + ci
            pltpu.sync_copy(src_hbm.at[row], sidx.at[0])
            pltpu.sync_copy(dst_hbm.at[row], didx.at[0])
            pltpu.sync_copy(w_hbm.at[row], wbuf)
            pltpu.async_copy(x_hbm.at[sidx.at[0]], rows, sem).wait()

            @pl.loop(0, _C)
            def _(e):
                wv = plsc.load_gather(wbuf, [jnp.full((16,), e, jnp.int32)])
                for kk in range(8):
                    sl = (e, pl.ds(kk * 16, 16))
                    rows[sl] = rows[sl] * wv

            pltpu.sync_copy(rows, acc.at[didx.at[0]], add=True)

        plsc.subcore_barrier()
        out_base = c * _NPAD + slab
        pltpu.sync_copy(acc.at[pl.ds(slab, _RPT)],
                        out_hbm.at[pl.ds(out_base, _RPT)])

    return k(src2, dst2, w2, x, zslab)


_BLK = 1000


def _tc_finish(x, partials, Wt, b2):
    def body(x_ref, p_ref, wt_ref, b_ref, o_ref):
        g = p_ref[0] + p_ref[1]
        nrm = jnp.sqrt(jnp.sum(g * g, axis=1, keepdims=True))
        gn = g / jnp.maximum(nrm, 1e-12)
        h = jnp.concatenate([x_ref[...], gn], axis=1)
        o = lax.dot_general(h, wt_ref[...], (((1,), (0,)), ((), ())),
                            preferred_element_type=jnp.float32,
                            precision=lax.Precision.HIGHEST) + b_ref[...]
        nrm2 = jnp.sqrt(jnp.sum(o * o, axis=1, keepdims=True))
        o_ref[...] = o / jnp.maximum(nrm2, 1e-12)

    return pl.pallas_call(
        body,
        grid=(_N // _BLK,),
        in_specs=[
            pl.BlockSpec((_BLK, _D), lambda i: (i, 0)),
            pl.BlockSpec((2, _BLK, _D), lambda i: (0, i, 0)),
            pl.BlockSpec((2 * _D, _D), lambda i: (0, 0)),
            pl.BlockSpec((1, _D), lambda i: (0, 0)),
        ],
        out_specs=pl.BlockSpec((_BLK, _D), lambda i: (i, 0)),
        out_shape=jax.ShapeDtypeStruct((_N, _D), jnp.float32),
    )(x, partials, Wt, b2)


def kernel(x, edge_index, edge_weight, W, b):
    src = edge_index[0]
    dst = edge_index[1]
    pad = _EPAD - _E
    src2 = jnp.pad(src, (0, pad)).reshape(_NTILES * _CPT, _C)
    dst2 = jnp.pad(dst, (0, pad)).reshape(_NTILES * _CPT, _C)
    w2 = jnp.pad(edge_weight, (0, pad)).reshape(_NTILES * _CPT, _C)
    zslab = jnp.zeros((_RPT, _D), jnp.float32)
    partials = _sc_scatter(src2, dst2, w2, x, zslab).reshape(2, _NPAD, _D)[:, :_N]
    return _tc_finish(x, partials, W.T, b[None, :])
